# Initial kernel scaffold; baseline (speedup 1.0000x reference)
import jax
import jax.numpy as jnp
from jax.experimental import pallas as pl


def kernel(input_x, table):
    raise NotImplementedError("write your pallas kernel here")


# speedup vs baseline: 13.8040x; 13.8040x over previous
import functools

import jax
import jax.numpy as jnp
from jax import lax
from jax.experimental import pallas as pl
from jax.experimental.pallas import tpu as pltpu
from jax.experimental.pallas import tpu_sc as plsc

EMB = 32
L = 200
NC = 2
NS = 16
NW = NC * NS
BS = 16
ROWS = BS * L
CHUNK = 128
NCH = ROWS // CHUNK
LANES = 16


def kernel(input_x, table):
    B = input_x.shape[0]
    assert input_x.shape[1] == L and table.shape[1] == EMB
    assert B % (NW * BS) == 0
    idx_flat = input_x.reshape(-1).astype(jnp.int32)
    S = B // NW
    NBLK = S // BS

    mesh = plsc.VectorSubcoreMesh(core_axis_name="c", subcore_axis_name="s")

    @functools.partial(
        pl.kernel,
        out_type=jax.ShapeDtypeStruct((B * EMB,), jnp.float32),
        mesh=mesh,
        scratch_types=[
            pltpu.VMEM((ROWS,), jnp.int32),
            pltpu.VMEM((ROWS, EMB), jnp.float32),
            pltpu.VMEM((BS * EMB,), jnp.float32),
            pltpu.SemaphoreType.DMA,
        ],
        compiler_params=pltpu.CompilerParams(
            needs_layout_passes=False, use_tc_tiling_on_sc=False),
    )
    def run(idx_hbm, table_hbm, out_hbm, idx_v, rows_v, res_v, sem):
        wid = lax.axis_index("s") * NC + lax.axis_index("c")
        base = wid * S
        lane = lax.broadcasted_iota(jnp.int32, (LANES,), 0)
        zeros = jnp.zeros((LANES,), jnp.float32)
        one_i = jnp.ones((LANES,), jnp.int32)
        zero_i = jnp.zeros((LANES,), jnp.int32)

        def block_body(b, carry):
            s0 = base + b * BS
            pltpu.sync_copy(idx_hbm.at[pl.ds(s0 * L, ROWS)], idx_v)

            def fire(j, c):
                pltpu.make_async_copy(
                    table_hbm.at[idx_v.at[pl.ds(j * CHUNK, CHUNK)]],
                    rows_v.at[pl.ds(j * CHUNK, CHUNK)],
                    sem).start()
                return c
            lax.fori_loop(0, NCH, fire, 0)

            ivec = lane * L

            def cnt_body(k, cnt):
                for j in range(8):
                    g = plsc.load_gather(idx_v, [ivec + (k * 8 + j)])
                    cnt = cnt + jnp.where(g != 0, one_i, zero_i)
                return cnt
            cnt = lax.fori_loop(0, L // 8, cnt_body, zero_i)
            cntf = cnt.astype(jnp.float32)

            def drain(j, c):
                pltpu.make_async_copy(
                    table_hbm.at[idx_v.at[pl.ds(j * CHUNK, CHUNK)]],
                    rows_v.at[pl.ds(j * CHUNK, CHUNK)],
                    sem).wait()
                return c
            lax.fori_loop(0, NCH, drain, 0)

            def sample_body(s, c):
                i0 = s * L

                def chunk(g, accs):
                    a0, a1, b0, b1 = accs
                    r = i0 + g * 8
                    for k in range(0, 8, 2):
                        a0 = a0 + rows_v[r + k, pl.ds(0, LANES)]
                        a1 = a1 + rows_v[r + k, pl.ds(LANES, LANES)]
                        b0 = b0 + rows_v[r + k + 1, pl.ds(0, LANES)]
                        b1 = b1 + rows_v[r + k + 1, pl.ds(LANES, LANES)]
                    return (a0, a1, b0, b1)

                a0, a1, b0, b1 = lax.fori_loop(
                    0, L // 8, chunk, (zeros, zeros, zeros, zeros))
                res_v[pl.ds(s * EMB, LANES)] = a0 + b0
                res_v[pl.ds(s * EMB + LANES, LANES)] = a1 + b1
                return c
            lax.fori_loop(0, BS, sample_body, 0)

            tcol = lane * EMB
            for e in range(EMB):
                col = plsc.load_gather(res_v, [tcol + e])
                plsc.store_scatter(res_v, [tcol + e], col / cntf)

            pltpu.sync_copy(res_v, out_hbm.at[pl.ds(s0 * EMB, BS * EMB)])
            return carry
        lax.fori_loop(0, NBLK, block_body, 0)

    return run(idx_flat, table).reshape(B, EMB)


# speedup vs baseline: 15.1089x; 1.0945x over previous
import functools

import jax
import jax.numpy as jnp
from jax import lax
from jax.experimental import pallas as pl
from jax.experimental.pallas import tpu as pltpu
from jax.experimental.pallas import tpu_sc as plsc

EMB = 32
L = 200
NC = 2
NS = 16
NW = NC * NS
BS = 16
ROWS = BS * L
G = 40
NG = L // G
LANES = 16


def kernel(input_x, table):
    B = input_x.shape[0]
    assert input_x.shape[1] == L and table.shape[1] == EMB
    assert B % (NW * BS) == 0
    idx_flat = input_x.reshape(-1).astype(jnp.int32)
    S = B // NW
    NBLK = S // BS

    mesh = plsc.VectorSubcoreMesh(core_axis_name="c", subcore_axis_name="s")

    @functools.partial(
        pl.kernel,
        out_type=jax.ShapeDtypeStruct((B * EMB,), jnp.float32),
        mesh=mesh,
        scratch_types=[
            pltpu.VMEM((ROWS,), jnp.int32),
            pltpu.VMEM((BS, G, EMB), jnp.float32),
            pltpu.VMEM((BS * EMB,), jnp.float32),
            pltpu.SemaphoreType.DMA,
            pltpu.SemaphoreType.DMA,
        ],
        compiler_params=pltpu.CompilerParams(
            needs_layout_passes=False, use_tc_tiling_on_sc=False),
    )
    def run(idx_hbm, table_hbm, out_hbm, idx_v, acc_v, res_v, semA, semB):
        wid = lax.axis_index("s") * NC + lax.axis_index("c")
        base = wid * S
        lane = lax.broadcasted_iota(jnp.int32, (LANES,), 0)
        zeros = jnp.zeros((LANES,), jnp.float32)
        one_i = jnp.ones((LANES,), jnp.int32)
        zero_i = jnp.zeros((LANES,), jnp.int32)

        def block_body(b, carry):
            s0 = base + b * BS
            pltpu.sync_copy(idx_hbm.at[pl.ds(s0 * L, ROWS)], idx_v)

            def fireA(s, c):
                pltpu.async_copy(
                    table_hbm.at[idx_v.at[pl.ds(s * L, G)]],
                    acc_v.at[s], semA)
                return c
            lax.fori_loop(0, BS, fireA, 0)

            ivec = lane * L

            def cnt_body(k, cnt):
                for j in range(8):
                    g = plsc.load_gather(idx_v, [ivec + (k * 8 + j)])
                    cnt = cnt + jnp.where(g != 0, one_i, zero_i)
                return cnt
            cnt = lax.fori_loop(0, L // 8, cnt_body, zero_i)
            cntf = cnt.astype(jnp.float32)

            def drainA(s, c):
                pltpu.make_async_copy(
                    table_hbm.at[idx_v.at[pl.ds(s * L, G)]],
                    acc_v.at[s], semA).wait()
                return c
            lax.fori_loop(0, BS, drainA, 0)

            def fireB(s, c):
                for k in range(1, NG):
                    pltpu.async_copy(
                        table_hbm.at[idx_v.at[pl.ds(s * L + k * G, G)]],
                        acc_v.at[s], semB, add=True)
                return c
            lax.fori_loop(0, BS, fireB, 0)

            def drainB(s, c):
                for k in range(1, NG):
                    pltpu.make_async_copy(
                        table_hbm.at[idx_v.at[pl.ds(s * L + k * G, G)]],
                        acc_v.at[s], semB).wait()
                return c
            lax.fori_loop(0, BS, drainB, 0)

            def sample_body(s, c):
                def chunk(g, accs):
                    a0, a1, b0, b1 = accs
                    r = g * 8
                    for k in range(0, 8, 2):
                        a0 = a0 + acc_v[s, r + k, pl.ds(0, LANES)]
                        a1 = a1 + acc_v[s, r + k, pl.ds(LANES, LANES)]
                        b0 = b0 + acc_v[s, r + k + 1, pl.ds(0, LANES)]
                        b1 = b1 + acc_v[s, r + k + 1, pl.ds(LANES, LANES)]
                    return (a0, a1, b0, b1)

                a0, a1, b0, b1 = lax.fori_loop(
                    0, G // 8, chunk, (zeros, zeros, zeros, zeros))
                res_v[pl.ds(s * EMB, LANES)] = a0 + b0
                res_v[pl.ds(s * EMB + LANES, LANES)] = a1 + b1
                return c
            lax.fori_loop(0, BS, sample_body, 0)

            tcol = lane * EMB
            for e in range(EMB):
                col = plsc.load_gather(res_v, [tcol + e])
                plsc.store_scatter(res_v, [tcol + e], col / cntf)

            pltpu.sync_copy(res_v, out_hbm.at[pl.ds(s0 * EMB, BS * EMB)])
            return carry
        lax.fori_loop(0, NBLK, block_body, 0)

    return run(idx_flat, table).reshape(B, EMB)


# speedup vs baseline: 16.1172x; 1.0667x over previous
import functools

import jax
import jax.numpy as jnp
from jax import lax
from jax.experimental import pallas as pl
from jax.experimental.pallas import tpu as pltpu
from jax.experimental.pallas import tpu_sc as plsc

EMB = 32
L = 200
NC = 2
NS = 16
NW = NC * NS
BS = 16
ROWS = BS * L
G = 40
NG = L // G
LANES = 16


def kernel(input_x, table):
    B = input_x.shape[0]
    assert input_x.shape[1] == L and table.shape[1] == EMB
    assert B % (NW * BS) == 0
    idx_flat = input_x.reshape(-1).astype(jnp.int32)
    S = B // NW
    NBLK = S // BS

    mesh = plsc.VectorSubcoreMesh(core_axis_name="c", subcore_axis_name="s")

    @functools.partial(
        pl.kernel,
        out_type=jax.ShapeDtypeStruct((B * EMB,), jnp.float32),
        mesh=mesh,
        scratch_types=[
            pltpu.VMEM((3, ROWS), jnp.int32),
            pltpu.VMEM((2, BS, G, EMB), jnp.float32),
            pltpu.VMEM((2, BS * EMB), jnp.float32),
            pltpu.VMEM((2, LANES), jnp.float32),
            pltpu.SemaphoreType.DMA,
            pltpu.SemaphoreType.DMA,
            pltpu.SemaphoreType.DMA,
            pltpu.SemaphoreType.DMA,
            pltpu.SemaphoreType.DMA,
        ],
        compiler_params=pltpu.CompilerParams(
            needs_layout_passes=False, use_tc_tiling_on_sc=False),
    )
    def run(idx_hbm, table_hbm, out_hbm, idx_v, acc_v, res_v, cnt_v,
            sem_idx, semA, semB0, semB1, sem_out):
        wid = lax.axis_index("s") * NC + lax.axis_index("c")
        base = wid * S
        lane = lax.broadcasted_iota(jnp.int32, (LANES,), 0)
        zeros = jnp.zeros((LANES,), jnp.float32)
        one_i = jnp.ones((LANES,), jnp.int32)
        zero_i = jnp.zeros((LANES,), jnp.int32)
        ivec = lane * L

        def stage(b):
            pltpu.async_copy(
                idx_hbm.at[pl.ds((base + b * BS) * L, ROWS)],
                idx_v.at[b % 3], sem_idx)

        def fire(b):
            ib = idx_v.at[b % 3]
            ab = acc_v.at[b % 2]
            pltpu.make_async_copy(
                idx_hbm.at[pl.ds((base + b * BS) * L, ROWS)],
                ib, sem_idx).wait()

            def fireA(s, c):
                pltpu.async_copy(
                    table_hbm.at[ib.at[pl.ds(s * L, G)]], ab.at[s], semA)
                return c
            lax.fori_loop(0, BS, fireA, 0)

            def cnt_body(k, cnt):
                for j in range(8):
                    g = plsc.load_gather(ib, [ivec + (k * 8 + j)])
                    cnt = cnt + jnp.where(g != 0, one_i, zero_i)
                return cnt
            cnt = lax.fori_loop(0, L // 8, cnt_body, zero_i)
            cnt_v[b % 2, pl.ds(0, LANES)] = cnt.astype(jnp.float32)

            def drainA(s, c):
                pltpu.make_async_copy(
                    table_hbm.at[ib.at[pl.ds(s * L, G)]], ab.at[s],
                    semA).wait()
                return c
            lax.fori_loop(0, BS, drainA, 0)

            def fireB(sem):
                def body(s, c):
                    for k in range(1, NG):
                        pltpu.async_copy(
                            table_hbm.at[ib.at[pl.ds(s * L + k * G, G)]],
                            ab.at[s], sem, add=True)
                    return c
                lax.fori_loop(0, BS, body, 0)

            @pl.when(b % 2 == 0)
            def _():
                fireB(semB0)

            @pl.when(b % 2 == 1)
            def _():
                fireB(semB1)

        def compute(b):
            ab = acc_v.at[b % 2]
            rb = res_v.at[b % 2]

            def drainB(sem):
                def body(s, c):
                    for k in range(1, NG):
                        pltpu.make_async_copy(
                            table_hbm.at[idx_v.at[b % 3].at[
                                pl.ds(s * L + k * G, G)]],
                            ab.at[s], sem).wait()
                    return c
                lax.fori_loop(0, BS, body, 0)

            @pl.when(b % 2 == 0)
            def _():
                drainB(semB0)

            @pl.when(b % 2 == 1)
            def _():
                drainB(semB1)

            @pl.when(b >= 2)
            def _():
                pltpu.make_async_copy(
                    res_v.at[b % 2],
                    out_hbm.at[pl.ds((base + (b - 2) * BS) * EMB, BS * EMB)],
                    sem_out).wait()

            def sample_body(s, c):
                def chunk(g, accs):
                    a0, a1, b0, b1 = accs
                    r = g * 8
                    for k in range(0, 8, 2):
                        a0 = a0 + ab[s, r + k, pl.ds(0, LANES)]
                        a1 = a1 + ab[s, r + k, pl.ds(LANES, LANES)]
                        b0 = b0 + ab[s, r + k + 1, pl.ds(0, LANES)]
                        b1 = b1 + ab[s, r + k + 1, pl.ds(LANES, LANES)]
                    return (a0, a1, b0, b1)

                a0, a1, b0, b1 = lax.fori_loop(
                    0, G // 8, chunk, (zeros, zeros, zeros, zeros))
                rb[pl.ds(s * EMB, LANES)] = a0 + b0
                rb[pl.ds(s * EMB + LANES, LANES)] = a1 + b1
                return c
            lax.fori_loop(0, BS, sample_body, 0)

            cntf = cnt_v[b % 2, pl.ds(0, LANES)]
            tcol = lane * EMB
            for e in range(EMB):
                col = plsc.load_gather(rb, [tcol + e])
                plsc.store_scatter(rb, [tcol + e], col / cntf)

            pltpu.async_copy(
                rb, out_hbm.at[pl.ds((base + b * BS) * EMB, BS * EMB)],
                sem_out)

        stage(0)
        stage(1)
        fire(0)

        def iter_body(b, c):
            @pl.when(b + 2 < NBLK)
            def _():
                stage(b + 2)

            @pl.when(b + 1 < NBLK)
            def _():
                fire(b + 1)
            compute(b)
            return c
        lax.fori_loop(0, NBLK, iter_body, 0)

        pltpu.make_async_copy(
            res_v.at[0],
            out_hbm.at[pl.ds((base + (NBLK - 2) * BS) * EMB, BS * EMB)],
            sem_out).wait()
        pltpu.make_async_copy(
            res_v.at[0],
            out_hbm.at[pl.ds((base + (NBLK - 1) * BS) * EMB, BS * EMB)],
            sem_out).wait()

    return run(idx_flat, table).reshape(B, EMB)


# speedup vs baseline: 16.5090x; 1.0243x over previous
import functools

import jax
import jax.numpy as jnp
from jax import lax
from jax.experimental import pallas as pl
from jax.experimental.pallas import tpu as pltpu
from jax.experimental.pallas import tpu_sc as plsc

EMB = 32
L = 200
NC = 2
NS = 16
NW = NC * NS
BS = 16
ROWS = BS * L
G = 40
NG = L // G
LANES = 16


def kernel(input_x, table):
    B = input_x.shape[0]
    assert input_x.shape[1] == L and table.shape[1] == EMB
    assert B % (NW * BS) == 0
    idx_flat = input_x.reshape(-1).astype(jnp.int32)
    S = B // NW
    NBLK = S // BS

    mesh = plsc.VectorSubcoreMesh(core_axis_name="c", subcore_axis_name="s")

    @functools.partial(
        pl.kernel,
        out_type=jax.ShapeDtypeStruct((B * EMB,), jnp.float32),
        mesh=mesh,
        scratch_types=[
            pltpu.VMEM((4, ROWS), jnp.int32),
            pltpu.VMEM((3, BS, G, EMB), jnp.float32),
            pltpu.VMEM((2, BS * EMB), jnp.float32),
            pltpu.VMEM((4, LANES), jnp.float32),
            pltpu.SemaphoreType.DMA,
            pltpu.SemaphoreType.DMA,
            pltpu.SemaphoreType.DMA,
            pltpu.SemaphoreType.DMA,
            pltpu.SemaphoreType.DMA,
            pltpu.SemaphoreType.DMA,
        ],
        compiler_params=pltpu.CompilerParams(
            needs_layout_passes=False, use_tc_tiling_on_sc=False),
    )
    def run(idx_hbm, table_hbm, out_hbm, idx_v, acc_v, res_v, cnt_v,
            sem_idx, semA0, semA1, semB0, semB1, sem_out):
        wid = lax.axis_index("s") * NC + lax.axis_index("c")
        base = wid * S
        lane = lax.broadcasted_iota(jnp.int32, (LANES,), 0)
        zeros = jnp.zeros((LANES,), jnp.float32)
        one_i = jnp.ones((LANES,), jnp.int32)
        zero_i = jnp.zeros((LANES,), jnp.int32)
        ivec = lane * L

        def stage(b):
            pltpu.async_copy(
                idx_hbm.at[pl.ds((base + b * BS) * L, ROWS)],
                idx_v.at[b % 4], sem_idx)

        def fireA(b):
            ib = idx_v.at[b % 4]
            ab = acc_v.at[b % 3]
            pltpu.make_async_copy(
                idx_hbm.at[pl.ds((base + b * BS) * L, ROWS)],
                ib, sem_idx).wait()

            def phaseA(sem):
                def body(s, c):
                    pltpu.async_copy(
                        table_hbm.at[ib.at[pl.ds(s * L, G)]], ab.at[s], sem)
                    return c
                lax.fori_loop(0, BS, body, 0)

            @pl.when(b % 2 == 0)
            def _():
                phaseA(semA0)

            @pl.when(b % 2 == 1)
            def _():
                phaseA(semA1)

            def cnt_body(k, cnt):
                for j in range(8):
                    g = plsc.load_gather(ib, [ivec + (k * 8 + j)])
                    cnt = cnt + jnp.where(g != 0, one_i, zero_i)
                return cnt
            cnt = lax.fori_loop(0, L // 8, cnt_body, zero_i)
            cnt_v[b % 4, pl.ds(0, LANES)] = cnt.astype(jnp.float32)

        def fireB(b):
            ib = idx_v.at[b % 4]
            ab = acc_v.at[b % 3]

            def drainA(sem):
                def body(s, c):
                    pltpu.make_async_copy(
                        table_hbm.at[ib.at[pl.ds(s * L, G)]], ab.at[s],
                        sem).wait()
                    return c
                lax.fori_loop(0, BS, body, 0)

            @pl.when(b % 2 == 0)
            def _():
                drainA(semA0)

            @pl.when(b % 2 == 1)
            def _():
                drainA(semA1)

            def phaseB(sem):
                def body(s, c):
                    for k in range(1, NG):
                        pltpu.async_copy(
                            table_hbm.at[ib.at[pl.ds(s * L + k * G, G)]],
                            ab.at[s], sem, add=True)
                    return c
                lax.fori_loop(0, BS, body, 0)

            @pl.when(b % 2 == 0)
            def _():
                phaseB(semB0)

            @pl.when(b % 2 == 1)
            def _():
                phaseB(semB1)

        def compute(b):
            ib = idx_v.at[b % 4]
            ab = acc_v.at[b % 3]
            rb = res_v.at[b % 2]

            def drainB(sem):
                def body(s, c):
                    for k in range(1, NG):
                        pltpu.make_async_copy(
                            table_hbm.at[ib.at[pl.ds(s * L + k * G, G)]],
                            ab.at[s], sem).wait()
                    return c
                lax.fori_loop(0, BS, body, 0)

            @pl.when(b % 2 == 0)
            def _():
                drainB(semB0)

            @pl.when(b % 2 == 1)
            def _():
                drainB(semB1)

            @pl.when(b >= 2)
            def _():
                pltpu.make_async_copy(
                    res_v.at[b % 2],
                    out_hbm.at[pl.ds((base + (b - 2) * BS) * EMB, BS * EMB)],
                    sem_out).wait()

            def sample_body(s, c):
                a0, a1, b0, b1 = zeros, zeros, zeros, zeros
                for r in range(0, G, 2):
                    a0 = a0 + ab[s, r, pl.ds(0, LANES)]
                    a1 = a1 + ab[s, r, pl.ds(LANES, LANES)]
                    b0 = b0 + ab[s, r + 1, pl.ds(0, LANES)]
                    b1 = b1 + ab[s, r + 1, pl.ds(LANES, LANES)]
                rb[pl.ds(s * EMB, LANES)] = a0 + b0
                rb[pl.ds(s * EMB + LANES, LANES)] = a1 + b1
                return c
            lax.fori_loop(0, BS, sample_body, 0)

            cntf = cnt_v[b % 4, pl.ds(0, LANES)]
            tcol = lane * EMB
            for e in range(EMB):
                col = plsc.load_gather(rb, [tcol + e])
                plsc.store_scatter(rb, [tcol + e], col / cntf)

            pltpu.async_copy(
                rb, out_hbm.at[pl.ds((base + b * BS) * EMB, BS * EMB)],
                sem_out)

        stage(0)
        stage(1)
        stage(2)
        fireA(0)
        fireA(1)
        fireB(0)

        def iter_body(b, c):
            @pl.when(b + 3 < NBLK)
            def _():
                stage(b + 3)

            @pl.when(b + 2 < NBLK)
            def _():
                fireA(b + 2)

            @pl.when(b + 1 < NBLK)
            def _():
                fireB(b + 1)
            compute(b)
            return c
        lax.fori_loop(0, NBLK, iter_body, 0)

        pltpu.make_async_copy(
            res_v.at[0],
            out_hbm.at[pl.ds((base + (NBLK - 2) * BS) * EMB, BS * EMB)],
            sem_out).wait()
        pltpu.make_async_copy(
            res_v.at[0],
            out_hbm.at[pl.ds((base + (NBLK - 1) * BS) * EMB, BS * EMB)],
            sem_out).wait()

    return run(idx_flat, table).reshape(B, EMB)


# speedup vs baseline: 16.5385x; 1.0018x over previous
import functools

import jax
import jax.numpy as jnp
from jax import lax
from jax.experimental import pallas as pl
from jax.experimental.pallas import tpu as pltpu
from jax.experimental.pallas import tpu_sc as plsc

EMB = 32
L = 200
NC = 2
NS = 16
NW = NC * NS
BS = 16
ROWS = BS * L
G = 40
NG = L // G
LANES = 16


def kernel(input_x, table):
    B = input_x.shape[0]
    assert input_x.shape[1] == L and table.shape[1] == EMB
    assert B % (NW * BS) == 0
    idx_flat = input_x.reshape(-1).astype(jnp.int32)
    table = jax.lax.optimization_barrier(table.reshape(-1)).reshape(
        table.shape)
    S = B // NW
    NBLK = S // BS

    mesh = plsc.VectorSubcoreMesh(core_axis_name="c", subcore_axis_name="s")

    @functools.partial(
        pl.kernel,
        out_type=jax.ShapeDtypeStruct((B * EMB,), jnp.float32),
        mesh=mesh,
        scratch_types=[
            pltpu.VMEM((4, ROWS), jnp.int32),
            pltpu.VMEM((3, BS, G, EMB), jnp.float32),
            pltpu.VMEM((2, BS * EMB), jnp.float32),
            pltpu.VMEM((4, LANES), jnp.float32),
            pltpu.SemaphoreType.DMA,
            pltpu.SemaphoreType.DMA,
            pltpu.SemaphoreType.DMA,
            pltpu.SemaphoreType.DMA,
            pltpu.SemaphoreType.DMA,
            pltpu.SemaphoreType.DMA,
        ],
        compiler_params=pltpu.CompilerParams(
            needs_layout_passes=False, use_tc_tiling_on_sc=False),
    )
    def run(idx_hbm, table_hbm, out_hbm, idx_v, acc_v, res_v, cnt_v,
            sem_idx, semA0, semA1, semB0, semB1, sem_out):
        wid = lax.axis_index("s") * NC + lax.axis_index("c")
        base = wid * S
        lane = lax.broadcasted_iota(jnp.int32, (LANES,), 0)
        zeros = jnp.zeros((LANES,), jnp.float32)
        one_i = jnp.ones((LANES,), jnp.int32)
        zero_i = jnp.zeros((LANES,), jnp.int32)
        ivec = lane * L

        def stage(b):
            pltpu.async_copy(
                idx_hbm.at[pl.ds((base + b * BS) * L, ROWS)],
                idx_v.at[b % 4], sem_idx)

        def fireA(b):
            ib = idx_v.at[b % 4]
            ab = acc_v.at[b % 3]
            pltpu.make_async_copy(
                idx_hbm.at[pl.ds((base + b * BS) * L, ROWS)],
                ib, sem_idx).wait()

            def phaseA(sem):
                def body(s, c):
                    pltpu.async_copy(
                        table_hbm.at[ib.at[pl.ds(s * L, G)]], ab.at[s], sem)
                    return c
                lax.fori_loop(0, BS, body, 0)

            @pl.when(b % 2 == 0)
            def _():
                phaseA(semA0)

            @pl.when(b % 2 == 1)
            def _():
                phaseA(semA1)

            def cnt_body(k, cnt):
                for j in range(8):
                    g = plsc.load_gather(ib, [ivec + (k * 8 + j)])
                    cnt = cnt + jnp.where(g != 0, one_i, zero_i)
                return cnt
            cnt = lax.fori_loop(0, L // 8, cnt_body, zero_i)
            cnt_v[b % 4, pl.ds(0, LANES)] = cnt.astype(jnp.float32)

        def fireB(b):
            ib = idx_v.at[b % 4]
            ab = acc_v.at[b % 3]

            def drainA(sem):
                def body(s, c):
                    pltpu.make_async_copy(
                        table_hbm.at[ib.at[pl.ds(s * L, G)]], ab.at[s],
                        sem).wait()
                    return c
                lax.fori_loop(0, BS, body, 0)

            @pl.when(b % 2 == 0)
            def _():
                drainA(semA0)

            @pl.when(b % 2 == 1)
            def _():
                drainA(semA1)

            def phaseB(sem):
                def body(s, c):
                    for k in range(1, NG):
                        pltpu.async_copy(
                            table_hbm.at[ib.at[pl.ds(s * L + k * G, G)]],
                            ab.at[s], sem, add=True)
                    return c
                lax.fori_loop(0, BS, body, 0)

            @pl.when(b % 2 == 0)
            def _():
                phaseB(semB0)

            @pl.when(b % 2 == 1)
            def _():
                phaseB(semB1)

        def compute(b):
            ib = idx_v.at[b % 4]
            ab = acc_v.at[b % 3]
            rb = res_v.at[b % 2]

            def drainB(sem):
                def body(s, c):
                    for k in range(1, NG):
                        pltpu.make_async_copy(
                            table_hbm.at[ib.at[pl.ds(s * L + k * G, G)]],
                            ab.at[s], sem).wait()
                    return c
                lax.fori_loop(0, BS, body, 0)

            @pl.when(b % 2 == 0)
            def _():
                drainB(semB0)

            @pl.when(b % 2 == 1)
            def _():
                drainB(semB1)

            @pl.when(b >= 2)
            def _():
                pltpu.make_async_copy(
                    res_v.at[b % 2],
                    out_hbm.at[pl.ds((base + (b - 2) * BS) * EMB, BS * EMB)],
                    sem_out).wait()

            def sample_body(s, c):
                a0, a1, b0, b1 = zeros, zeros, zeros, zeros
                for r in range(0, G, 2):
                    a0 = a0 + ab[s, r, pl.ds(0, LANES)]
                    a1 = a1 + ab[s, r, pl.ds(LANES, LANES)]
                    b0 = b0 + ab[s, r + 1, pl.ds(0, LANES)]
                    b1 = b1 + ab[s, r + 1, pl.ds(LANES, LANES)]
                rb[pl.ds(s * EMB, LANES)] = a0 + b0
                rb[pl.ds(s * EMB + LANES, LANES)] = a1 + b1
                return c
            lax.fori_loop(0, BS, sample_body, 0)

            cntf = cnt_v[b % 4, pl.ds(0, LANES)]
            tcol = lane * EMB
            for e in range(EMB):
                col = plsc.load_gather(rb, [tcol + e])
                plsc.store_scatter(rb, [tcol + e], col / cntf)

            pltpu.async_copy(
                rb, out_hbm.at[pl.ds((base + b * BS) * EMB, BS * EMB)],
                sem_out)

        stage(0)
        stage(1)
        stage(2)
        fireA(0)
        fireA(1)
        fireB(0)

        def iter_body(b, c):
            @pl.when(b + 3 < NBLK)
            def _():
                stage(b + 3)

            @pl.when(b + 2 < NBLK)
            def _():
                fireA(b + 2)

            @pl.when(b + 1 < NBLK)
            def _():
                fireB(b + 1)
            compute(b)
            return c
        lax.fori_loop(0, NBLK, iter_body, 0)

        pltpu.make_async_copy(
            res_v.at[0],
            out_hbm.at[pl.ds((base + (NBLK - 2) * BS) * EMB, BS * EMB)],
            sem_out).wait()
        pltpu.make_async_copy(
            res_v.at[0],
            out_hbm.at[pl.ds((base + (NBLK - 1) * BS) * EMB, BS * EMB)],
            sem_out).wait()

    return run(idx_flat, table).reshape(B, EMB)


# speedup vs baseline: 16.6145x; 1.0046x over previous
import functools

import jax
import jax.numpy as jnp
from jax import lax
from jax.experimental import pallas as pl
from jax.experimental.pallas import tpu as pltpu
from jax.experimental.pallas import tpu_sc as plsc

EMB = 32
L = 200
NC = 2
NS = 16
NW = NC * NS
BS = 16
ROWS = BS * L
G = 40
NG = L // G
LANES = 16


def kernel(input_x, table):
    B = input_x.shape[0]
    assert input_x.shape[1] == L and table.shape[1] == EMB
    assert B % (NW * BS) == 0
    tbl4 = jnp.pad(table, ((0, 0), (0, 128 - EMB))).reshape(-1, EMB)
    idx_flat = input_x.reshape(-1).astype(jnp.int32) * 4
    S = B // NW
    NBLK = S // BS

    mesh = plsc.VectorSubcoreMesh(core_axis_name="c", subcore_axis_name="s")

    @functools.partial(
        pl.kernel,
        out_type=jax.ShapeDtypeStruct((B * EMB,), jnp.float32),
        mesh=mesh,
        scratch_types=[
            pltpu.VMEM((4, ROWS), jnp.int32),
            pltpu.VMEM((3, BS, G, EMB), jnp.float32),
            pltpu.VMEM((2, BS * EMB), jnp.float32),
            pltpu.VMEM((4, LANES), jnp.float32),
            pltpu.SemaphoreType.DMA,
            pltpu.SemaphoreType.DMA,
            pltpu.SemaphoreType.DMA,
            pltpu.SemaphoreType.DMA,
            pltpu.SemaphoreType.DMA,
            pltpu.SemaphoreType.DMA,
        ],
        compiler_params=pltpu.CompilerParams(
            needs_layout_passes=False, use_tc_tiling_on_sc=False),
    )
    def run(idx_hbm, table_hbm, out_hbm, idx_v, acc_v, res_v, cnt_v,
            sem_idx, semA0, semA1, semB0, semB1, sem_out):
        wid = lax.axis_index("s") * NC + lax.axis_index("c")
        base = wid * S
        lane = lax.broadcasted_iota(jnp.int32, (LANES,), 0)
        zeros = jnp.zeros((LANES,), jnp.float32)
        one_i = jnp.ones((LANES,), jnp.int32)
        zero_i = jnp.zeros((LANES,), jnp.int32)
        ivec = lane * L

        def stage(b):
            pltpu.async_copy(
                idx_hbm.at[pl.ds((base + b * BS) * L, ROWS)],
                idx_v.at[b % 4], sem_idx)

        def fireA(b):
            ib = idx_v.at[b % 4]
            ab = acc_v.at[b % 3]
            pltpu.make_async_copy(
                idx_hbm.at[pl.ds((base + b * BS) * L, ROWS)],
                ib, sem_idx).wait()

            def phaseA(sem):
                def body(s, c):
                    pltpu.async_copy(
                        table_hbm.at[ib.at[pl.ds(s * L, G)]], ab.at[s], sem)
                    return c
                lax.fori_loop(0, BS, body, 0)

            @pl.when(b % 2 == 0)
            def _():
                phaseA(semA0)

            @pl.when(b % 2 == 1)
            def _():
                phaseA(semA1)

            def cnt_body(k, cnt):
                for j in range(8):
                    g = plsc.load_gather(ib, [ivec + (k * 8 + j)])
                    cnt = cnt + jnp.where(g != 0, one_i, zero_i)
                return cnt
            cnt = lax.fori_loop(0, L // 8, cnt_body, zero_i)
            cnt_v[b % 4, pl.ds(0, LANES)] = cnt.astype(jnp.float32)

        def fireB(b):
            ib = idx_v.at[b % 4]
            ab = acc_v.at[b % 3]

            def drainA(sem):
                def body(s, c):
                    pltpu.make_async_copy(
                        table_hbm.at[ib.at[pl.ds(s * L, G)]], ab.at[s],
                        sem).wait()
                    return c
                lax.fori_loop(0, BS, body, 0)

            @pl.when(b % 2 == 0)
            def _():
                drainA(semA0)

            @pl.when(b % 2 == 1)
            def _():
                drainA(semA1)

            def phaseB(sem):
                def body(s, c):
                    for k in range(1, NG):
                        pltpu.async_copy(
                            table_hbm.at[ib.at[pl.ds(s * L + k * G, G)]],
                            ab.at[s], sem, add=True)
                    return c
                lax.fori_loop(0, BS, body, 0)

            @pl.when(b % 2 == 0)
            def _():
                phaseB(semB0)

            @pl.when(b % 2 == 1)
            def _():
                phaseB(semB1)

        def compute(b):
            ib = idx_v.at[b % 4]
            ab = acc_v.at[b % 3]
            rb = res_v.at[b % 2]

            def drainB(sem):
                def body(s, c):
                    for k in range(1, NG):
                        pltpu.make_async_copy(
                            table_hbm.at[ib.at[pl.ds(s * L + k * G, G)]],
                            ab.at[s], sem).wait()
                    return c
                lax.fori_loop(0, BS, body, 0)

            @pl.when(b % 2 == 0)
            def _():
                drainB(semB0)

            @pl.when(b % 2 == 1)
            def _():
                drainB(semB1)

            @pl.when(b >= 2)
            def _():
                pltpu.make_async_copy(
                    res_v.at[b % 2],
                    out_hbm.at[pl.ds((base + (b - 2) * BS) * EMB, BS * EMB)],
                    sem_out).wait()

            def sample_body(s, c):
                a0, a1, b0, b1 = zeros, zeros, zeros, zeros
                for r in range(0, G, 2):
                    a0 = a0 + ab[s, r, pl.ds(0, LANES)]
                    a1 = a1 + ab[s, r, pl.ds(LANES, LANES)]
                    b0 = b0 + ab[s, r + 1, pl.ds(0, LANES)]
                    b1 = b1 + ab[s, r + 1, pl.ds(LANES, LANES)]
                rb[pl.ds(s * EMB, LANES)] = a0 + b0
                rb[pl.ds(s * EMB + LANES, LANES)] = a1 + b1
                return c
            lax.fori_loop(0, BS, sample_body, 0)

            cntf = cnt_v[b % 4, pl.ds(0, LANES)]
            tcol = lane * EMB
            for e in range(EMB):
                col = plsc.load_gather(rb, [tcol + e])
                plsc.store_scatter(rb, [tcol + e], col / cntf)

            pltpu.async_copy(
                rb, out_hbm.at[pl.ds((base + b * BS) * EMB, BS * EMB)],
                sem_out)

        stage(0)
        stage(1)
        stage(2)
        fireA(0)
        fireA(1)
        fireB(0)

        def iter_body(b, c):
            @pl.when(b + 3 < NBLK)
            def _():
                stage(b + 3)

            @pl.when(b + 2 < NBLK)
            def _():
                fireA(b + 2)

            @pl.when(b + 1 < NBLK)
            def _():
                fireB(b + 1)
            compute(b)
            return c
        lax.fori_loop(0, NBLK, iter_body, 0)

        pltpu.make_async_copy(
            res_v.at[0],
            out_hbm.at[pl.ds((base + (NBLK - 2) * BS) * EMB, BS * EMB)],
            sem_out).wait()
        pltpu.make_async_copy(
            res_v.at[0],
            out_hbm.at[pl.ds((base + (NBLK - 1) * BS) * EMB, BS * EMB)],
            sem_out).wait()

    return run(idx_flat, tbl4).reshape(B, EMB)


# speedup vs baseline: 16.7980x; 1.0110x over previous
import functools

import jax
import jax.numpy as jnp
from jax import lax
from jax.experimental import pallas as pl
from jax.experimental.pallas import tpu as pltpu
from jax.experimental.pallas import tpu_sc as plsc

EMB = 32
L = 200
SROW = 256
NC = 2
NS = 16
NW = NC * NS
BS = 16
IDXB = BS * SROW
G = 40
NG = L // G
LANES = 16
PREPB = 256


def _prep_body(x_ref, idx_ref, len_ref):
    x = x_ref[...]
    idx_ref[...] = jnp.pad(x * 4, ((0, 0), (0, SROW - L)))
    len_ref[...] = jnp.sum((x != 0).astype(jnp.float32), axis=1)


def kernel(input_x, table):
    B = input_x.shape[0]
    assert input_x.shape[1] == L and table.shape[1] == EMB
    assert B % (NW * BS) == 0
    tbl4 = jnp.pad(table, ((0, 0), (0, 128 - EMB))).reshape(-1, EMB)

    idx4, lens = pl.pallas_call(
        _prep_body,
        grid=(B // PREPB,),
        in_specs=[pl.BlockSpec((PREPB, L), lambda i: (i, 0))],
        out_specs=[pl.BlockSpec((PREPB, SROW), lambda i: (i, 0)),
                   pl.BlockSpec((PREPB,), lambda i: (i,))],
        out_shape=[jax.ShapeDtypeStruct((B, SROW), jnp.int32),
                   jax.ShapeDtypeStruct((B,), jnp.float32)],
    )(input_x.astype(jnp.int32))
    idx_flat = idx4.reshape(-1)
    lens_flat = lens

    S = B // NW
    NBLK = S // BS

    mesh = plsc.VectorSubcoreMesh(core_axis_name="c", subcore_axis_name="s")

    @functools.partial(
        pl.kernel,
        out_type=jax.ShapeDtypeStruct((B * EMB,), jnp.float32),
        mesh=mesh,
        scratch_types=[
            pltpu.VMEM((4, IDXB), jnp.int32),
            pltpu.VMEM((3, BS, G, EMB), jnp.float32),
            pltpu.VMEM((2, BS * EMB), jnp.float32),
            pltpu.VMEM((4, LANES), jnp.float32),
            pltpu.SemaphoreType.DMA,
            pltpu.SemaphoreType.DMA,
            pltpu.SemaphoreType.DMA,
            pltpu.SemaphoreType.DMA,
            pltpu.SemaphoreType.DMA,
            pltpu.SemaphoreType.DMA,
        ],
        compiler_params=pltpu.CompilerParams(
            needs_layout_passes=False, use_tc_tiling_on_sc=False),
    )
    def run(idx_hbm, lens_hbm, table_hbm, out_hbm, idx_v, acc_v, res_v, cnt_v,
            sem_idx, semA0, semA1, semB0, semB1, sem_out):
        wid = lax.axis_index("s") * NC + lax.axis_index("c")
        base = wid * S
        lane = lax.broadcasted_iota(jnp.int32, (LANES,), 0)
        zeros = jnp.zeros((LANES,), jnp.float32)

        def stage(b):
            pltpu.async_copy(
                idx_hbm.at[pl.ds((base + b * BS) * SROW, IDXB)],
                idx_v.at[b % 4], sem_idx)
            pltpu.async_copy(
                lens_hbm.at[pl.ds(base + b * BS, BS)],
                cnt_v.at[b % 4], sem_idx)

        def fireA(b):
            ib = idx_v.at[b % 4]
            ab = acc_v.at[b % 3]
            pltpu.make_async_copy(
                idx_hbm.at[pl.ds((base + b * BS) * SROW, IDXB)],
                ib, sem_idx).wait()
            pltpu.make_async_copy(
                lens_hbm.at[pl.ds(base + b * BS, BS)],
                cnt_v.at[b % 4], sem_idx).wait()

            def phaseA(sem):
                def body(s, c):
                    pltpu.async_copy(
                        table_hbm.at[ib.at[pl.ds(s * SROW, G)]], ab.at[s],
                        sem)
                    return c
                lax.fori_loop(0, BS, body, 0)

            @pl.when(b % 2 == 0)
            def _():
                phaseA(semA0)

            @pl.when(b % 2 == 1)
            def _():
                phaseA(semA1)

        def fireB(b):
            ib = idx_v.at[b % 4]
            ab = acc_v.at[b % 3]

            def drainA(sem):
                def body(s, c):
                    pltpu.make_async_copy(
                        table_hbm.at[ib.at[pl.ds(s * SROW, G)]], ab.at[s],
                        sem).wait()
                    return c
                lax.fori_loop(0, BS, body, 0)

            @pl.when(b % 2 == 0)
            def _():
                drainA(semA0)

            @pl.when(b % 2 == 1)
            def _():
                drainA(semA1)

            def phaseB(sem):
                def body(s, c):
                    for k in range(1, NG):
                        pltpu.async_copy(
                            table_hbm.at[ib.at[pl.ds(s * SROW + k * G, G)]],
                            ab.at[s], sem, add=True)
                    return c
                lax.fori_loop(0, BS, body, 0)

            @pl.when(b % 2 == 0)
            def _():
                phaseB(semB0)

            @pl.when(b % 2 == 1)
            def _():
                phaseB(semB1)

        def compute(b):
            ib = idx_v.at[b % 4]
            ab = acc_v.at[b % 3]
            rb = res_v.at[b % 2]

            def drainB(sem):
                def body(s, c):
                    for k in range(1, NG):
                        pltpu.make_async_copy(
                            table_hbm.at[ib.at[pl.ds(s * SROW + k * G, G)]],
                            ab.at[s], sem).wait()
                    return c
                lax.fori_loop(0, BS, body, 0)

            @pl.when(b % 2 == 0)
            def _():
                drainB(semB0)

            @pl.when(b % 2 == 1)
            def _():
                drainB(semB1)

            @pl.when(b >= 2)
            def _():
                pltpu.make_async_copy(
                    res_v.at[b % 2],
                    out_hbm.at[pl.ds((base + (b - 2) * BS) * EMB, BS * EMB)],
                    sem_out).wait()

            def sample_body(s, c):
                a0, a1, b0, b1 = zeros, zeros, zeros, zeros
                for r in range(0, G, 2):
                    a0 = a0 + ab[s, r, pl.ds(0, LANES)]
                    a1 = a1 + ab[s, r, pl.ds(LANES, LANES)]
                    b0 = b0 + ab[s, r + 1, pl.ds(0, LANES)]
                    b1 = b1 + ab[s, r + 1, pl.ds(LANES, LANES)]
                rb[pl.ds(s * EMB, LANES)] = a0 + b0
                rb[pl.ds(s * EMB + LANES, LANES)] = a1 + b1
                return c
            lax.fori_loop(0, BS, sample_body, 0)

            cntf = cnt_v[b % 4, pl.ds(0, LANES)]
            tcol = lane * EMB
            for e in range(EMB):
                col = plsc.load_gather(rb, [tcol + e])
                plsc.store_scatter(rb, [tcol + e], col / cntf)

            pltpu.async_copy(
                rb, out_hbm.at[pl.ds((base + b * BS) * EMB, BS * EMB)],
                sem_out)

        stage(0)
        stage(1)
        stage(2)
        fireA(0)
        fireA(1)
        fireB(0)

        def iter_body(b, c):
            @pl.when(b + 3 < NBLK)
            def _():
                stage(b + 3)

            @pl.when(b + 2 < NBLK)
            def _():
                fireA(b + 2)

            @pl.when(b + 1 < NBLK)
            def _():
                fireB(b + 1)
            compute(b)
            return c
        lax.fori_loop(0, NBLK, iter_body, 0)

        pltpu.make_async_copy(
            res_v.at[0],
            out_hbm.at[pl.ds((base + (NBLK - 2) * BS) * EMB, BS * EMB)],
            sem_out).wait()
        pltpu.make_async_copy(
            res_v.at[0],
            out_hbm.at[pl.ds((base + (NBLK - 1) * BS) * EMB, BS * EMB)],
            sem_out).wait()

    return run(idx_flat, lens_flat, tbl4).reshape(B, EMB)
